# baseline (device time: 1520944 ns/iter reference)
import jax
import jax.numpy as jnp
from jax import lax
from jax.experimental import pallas as pl
from jax.experimental.pallas import tpu as pltpu

N_DEV = 16


def kernel(x, w_mat):
    m, k_per = x.shape
    _, n = w_mat.shape
    chunk = m // N_DEV

    def body(x_ref, w_ref, out_ref, comm_ref, amax_ref,
             send_sems, recv_sems, amax_send_sems, amax_recv_sems):
        my = lax.axis_index("i")
        left = lax.rem(my - 1 + N_DEV, N_DEV)
        right = lax.rem(my + 1, N_DEV)

        barrier_sem = pltpu.get_barrier_semaphore()
        pl.semaphore_signal(barrier_sem, inc=1, device_id=(left,),
                            device_id_type=pl.DeviceIdType.MESH)
        pl.semaphore_signal(barrier_sem, inc=1, device_id=(right,),
                            device_id_type=pl.DeviceIdType.MESH)
        pl.semaphore_wait(barrier_sem, 2)

        def partial_for(c):
            return jnp.dot(
                x_ref[pl.ds(c * chunk, chunk), :], w_ref[...],
                preferred_element_type=jnp.float32,
                precision=lax.Precision.HIGHEST,
            )

        for s in range(N_DEV - 1):
            c = lax.rem(my - 1 - s + 2 * N_DEV, N_DEV)
            part = partial_for(c)
            send_slot = s % 2
            recv_slot = (s + 1) % 2
            if s == 0:
                comm_ref[send_slot] = part
            else:
                comm_ref[send_slot] = comm_ref[send_slot] + part
            rdma = pltpu.make_async_remote_copy(
                src_ref=comm_ref.at[send_slot],
                dst_ref=comm_ref.at[recv_slot],
                send_sem=send_sems.at[send_slot],
                recv_sem=recv_sems.at[recv_slot],
                device_id=(right,),
                device_id_type=pl.DeviceIdType.MESH,
            )
            rdma.start()
            rdma.wait()

        y = comm_ref[1] + partial_for(my)

        local_amax = jnp.max(jnp.abs(y))
        amax_ref[pl.ds(my, 1), :] = jnp.full((1, 128), local_amax, jnp.float32)

        sends = []
        for d in range(1, N_DEV):
            dest = lax.rem(my + d, N_DEV)
            snd = pltpu.make_async_remote_copy(
                src_ref=amax_ref.at[pl.ds(my, 1)],
                dst_ref=amax_ref.at[pl.ds(my, 1)],
                send_sem=amax_send_sems.at[d],
                recv_sem=amax_recv_sems.at[d],
                device_id=(dest,),
                device_id_type=pl.DeviceIdType.MESH,
            )
            snd.start()
            sends.append(snd)
        for d in range(1, N_DEV):
            src_dev = lax.rem(my - d + 2 * N_DEV, N_DEV)
            rcv = pltpu.make_async_remote_copy(
                src_ref=amax_ref.at[pl.ds(my, 1)],
                dst_ref=amax_ref.at[pl.ds(src_dev, 1)],
                send_sem=amax_send_sems.at[d],
                recv_sem=amax_recv_sems.at[d],
                device_id=(src_dev,),
                device_id_type=pl.DeviceIdType.MESH,
            )
            rcv.wait_recv()
        for snd in sends:
            snd.wait_send()

        gmax = jnp.max(amax_ref[...])
        scale = gmax / 127.0
        q = jnp.clip(jnp.round(y / scale), -127.0, 127.0)
        out_ref[...] = q * scale

    return pl.pallas_call(
        body,
        out_shape=jax.ShapeDtypeStruct((chunk, n), jnp.float32),
        in_specs=[
            pl.BlockSpec(memory_space=pltpu.VMEM),
            pl.BlockSpec(memory_space=pltpu.VMEM),
        ],
        out_specs=pl.BlockSpec(memory_space=pltpu.VMEM),
        scratch_shapes=[
            pltpu.VMEM((2, chunk, n), jnp.float32),
            pltpu.VMEM((N_DEV, 128), jnp.float32),
            pltpu.SemaphoreType.DMA((2,)),
            pltpu.SemaphoreType.DMA((2,)),
            pltpu.SemaphoreType.DMA((N_DEV,)),
            pltpu.SemaphoreType.DMA((N_DEV,)),
        ],
        compiler_params=pltpu.CompilerParams(collective_id=0),
    )(x, w_mat)


# device time: 763937 ns/iter; 1.9909x vs baseline; 1.9909x over previous
import jax
import jax.numpy as jnp
from jax import lax
from jax.experimental import pallas as pl
from jax.experimental.pallas import tpu as pltpu

N_DEV = 16
N_SLOTS = 3


def kernel(x, w_mat):
    m, k_per = x.shape
    _, n = w_mat.shape
    chunk = m // N_DEV
    n2 = n // 2

    def body(x_ref, w_ref, out_ref, cw_ref, ccw_ref, amax_ref,
             cw_send_sems, cw_recv_sems, ccw_send_sems, ccw_recv_sems,
             amax_send_sems, amax_recv_sems):
        my = lax.axis_index("i")
        left = lax.rem(my - 1 + N_DEV, N_DEV)
        right = lax.rem(my + 1, N_DEV)

        barrier_sem = pltpu.get_barrier_semaphore()
        pl.semaphore_signal(barrier_sem, inc=1, device_id=(left,),
                            device_id_type=pl.DeviceIdType.MESH)
        pl.semaphore_signal(barrier_sem, inc=1, device_id=(right,),
                            device_id_type=pl.DeviceIdType.MESH)
        pl.semaphore_wait(barrier_sem, 2)

        def part_cw(c):
            return jnp.dot(
                x_ref[pl.ds(c * chunk, chunk), :], w_ref[:, :n2],
                preferred_element_type=jnp.float32,
                precision=lax.Precision.HIGHEST,
            )

        def part_ccw(c):
            return jnp.dot(
                x_ref[pl.ds(c * chunk, chunk), :], w_ref[:, n2:],
                preferred_element_type=jnp.float32,
                precision=lax.Precision.HIGHEST,
            )

        def c_cw(s):
            return lax.rem(my - 1 - s + 2 * N_DEV, N_DEV)

        def c_ccw(s):
            return lax.rem(my + 1 + s, N_DEV)

        cw_ref[0] = part_cw(c_cw(0))
        ccw_ref[0] = part_ccw(c_ccw(0))

        prev = None
        for s in range(N_DEV - 1):
            slot = s % N_SLOTS
            rslot = (s + 1) % N_SLOTS
            if prev is not None:
                prev[0].wait_send()
                prev[1].wait_send()
            cw = pltpu.make_async_remote_copy(
                src_ref=cw_ref.at[slot],
                dst_ref=cw_ref.at[rslot],
                send_sem=cw_send_sems.at[slot],
                recv_sem=cw_recv_sems.at[rslot],
                device_id=(right,),
                device_id_type=pl.DeviceIdType.MESH,
            )
            ccw = pltpu.make_async_remote_copy(
                src_ref=ccw_ref.at[slot],
                dst_ref=ccw_ref.at[rslot],
                send_sem=ccw_send_sems.at[slot],
                recv_sem=ccw_recv_sems.at[rslot],
                device_id=(left,),
                device_id_type=pl.DeviceIdType.MESH,
            )
            cw.start()
            ccw.start()
            pn_cw = part_cw(c_cw(s + 1))
            pn_ccw = part_ccw(c_ccw(s + 1))
            cw.wait_recv()
            ccw.wait_recv()
            if s < N_DEV - 2:
                cw_ref[rslot] = cw_ref[rslot] + pn_cw
                ccw_ref[rslot] = ccw_ref[rslot] + pn_ccw
            else:
                out_ref[:, :n2] = cw_ref[rslot] + pn_cw
                out_ref[:, n2:] = ccw_ref[rslot] + pn_ccw
            prev = (cw, ccw)
        prev[0].wait_send()
        prev[1].wait_send()

        local_amax = jnp.max(jnp.abs(out_ref[...]))
        amax_ref[pl.ds(my, 1), :] = jnp.full((1, 128), local_amax, jnp.float32)

        sends = []
        for d in range(1, N_DEV):
            dest = lax.rem(my + d, N_DEV)
            snd = pltpu.make_async_remote_copy(
                src_ref=amax_ref.at[pl.ds(my, 1)],
                dst_ref=amax_ref.at[pl.ds(my, 1)],
                send_sem=amax_send_sems.at[d],
                recv_sem=amax_recv_sems.at[d],
                device_id=(dest,),
                device_id_type=pl.DeviceIdType.MESH,
            )
            snd.start()
            sends.append(snd)
        for d in range(1, N_DEV):
            src_dev = lax.rem(my - d + 2 * N_DEV, N_DEV)
            rcv = pltpu.make_async_remote_copy(
                src_ref=amax_ref.at[pl.ds(my, 1)],
                dst_ref=amax_ref.at[pl.ds(src_dev, 1)],
                send_sem=amax_send_sems.at[d],
                recv_sem=amax_recv_sems.at[d],
                device_id=(src_dev,),
                device_id_type=pl.DeviceIdType.MESH,
            )
            rcv.wait_recv()
        for snd in sends:
            snd.wait_send()

        gmax = jnp.max(amax_ref[...])
        scale = gmax / 127.0
        q = jnp.clip(jnp.round(out_ref[...] / scale), -127.0, 127.0)
        out_ref[...] = q * scale

    return pl.pallas_call(
        body,
        out_shape=jax.ShapeDtypeStruct((chunk, n), jnp.float32),
        in_specs=[
            pl.BlockSpec(memory_space=pltpu.VMEM),
            pl.BlockSpec(memory_space=pltpu.VMEM),
        ],
        out_specs=pl.BlockSpec(memory_space=pltpu.VMEM),
        scratch_shapes=[
            pltpu.VMEM((N_SLOTS, chunk, n2), jnp.float32),
            pltpu.VMEM((N_SLOTS, chunk, n2), jnp.float32),
            pltpu.VMEM((N_DEV, 128), jnp.float32),
            pltpu.SemaphoreType.DMA((N_SLOTS,)),
            pltpu.SemaphoreType.DMA((N_SLOTS,)),
            pltpu.SemaphoreType.DMA((N_SLOTS,)),
            pltpu.SemaphoreType.DMA((N_SLOTS,)),
            pltpu.SemaphoreType.DMA((N_DEV,)),
            pltpu.SemaphoreType.DMA((N_DEV,)),
        ],
        compiler_params=pltpu.CompilerParams(
            collective_id=0,
            vmem_limit_bytes=60 * 1024 * 1024,
        ),
    )(x, w_mat)


# device time: 709134 ns/iter; 2.1448x vs baseline; 1.0773x over previous
import jax
import jax.numpy as jnp
from jax import lax
from jax.experimental import pallas as pl
from jax.experimental.pallas import tpu as pltpu

N_DEV = 16
N_SLOTS = 3
N_SUB = 2


def kernel(x, w_mat):
    m, k_per = x.shape
    _, n = w_mat.shape
    chunk = m // N_DEV
    n2 = n // 2
    nsub = n2 // N_SUB

    def body(x_ref, w_ref, out_ref, cw_ref, ccw_ref, amax_ref,
             cw_send_sems, cw_recv_sems, ccw_send_sems, ccw_recv_sems,
             amax_send_sems, amax_recv_sems):
        my = lax.axis_index("i")
        left = lax.rem(my - 1 + N_DEV, N_DEV)
        right = lax.rem(my + 1, N_DEV)

        barrier_sem = pltpu.get_barrier_semaphore()
        pl.semaphore_signal(barrier_sem, inc=1, device_id=(left,),
                            device_id_type=pl.DeviceIdType.MESH)
        pl.semaphore_signal(barrier_sem, inc=1, device_id=(right,),
                            device_id_type=pl.DeviceIdType.MESH)
        pl.semaphore_wait(barrier_sem, 2)

        def part_cw(c):
            return jnp.dot(
                x_ref[pl.ds(c * chunk, chunk), :], w_ref[:, :n2],
                preferred_element_type=jnp.float32,
                precision=lax.Precision.HIGHEST,
            )

        def part_ccw(c):
            return jnp.dot(
                x_ref[pl.ds(c * chunk, chunk), :], w_ref[:, n2:],
                preferred_element_type=jnp.float32,
                precision=lax.Precision.HIGHEST,
            )

        def c_cw(s):
            return lax.rem(my - 1 - s + 2 * N_DEV, N_DEV)

        def c_ccw(s):
            return lax.rem(my + 1 + s, N_DEV)

        def make_desc(ref, slot, sub, send_sems, recv_sems, dest):
            return pltpu.make_async_remote_copy(
                src_ref=ref.at[slot, sub],
                dst_ref=ref.at[(slot + 1) % N_SLOTS, sub],
                send_sem=send_sems.at[slot * N_SUB + sub],
                recv_sem=recv_sems.at[((slot + 1) % N_SLOTS) * N_SUB + sub],
                device_id=(dest,),
                device_id_type=pl.DeviceIdType.MESH,
            )

        p_cw = part_cw(c_cw(0))
        p_ccw = part_ccw(c_ccw(0))
        sends0 = []
        for sub in range(N_SUB):
            lo = sub * nsub
            cw_ref[0, sub] = p_cw[:, lo:lo + nsub]
            ccw_ref[0, sub] = p_ccw[:, lo:lo + nsub]
            d_cw = make_desc(cw_ref, 0, sub, cw_send_sems, cw_recv_sems, right)
            d_ccw = make_desc(ccw_ref, 0, sub, ccw_send_sems, ccw_recv_sems,
                              left)
            d_cw.start()
            d_ccw.start()
            sends0.append((d_cw, d_ccw))
        sends = {0: sends0}
        pn_cw = part_cw(c_cw(1))
        pn_ccw = part_ccw(c_ccw(1))

        for s in range(N_DEV - 2):
            rslot = (s + 1) % N_SLOTS
            if s - 2 in sends:
                for d_cw, d_ccw in sends.pop(s - 2):
                    d_cw.wait_send()
                    d_ccw.wait_send()
            step_descs = []
            for sub in range(N_SUB):
                lo = sub * nsub
                sends[s][sub][0].wait_recv()
                cw_ref[rslot, sub] = cw_ref[rslot, sub] + pn_cw[:, lo:lo + nsub]
                d_cw = make_desc(cw_ref, rslot, sub, cw_send_sems,
                                 cw_recv_sems, right)
                d_cw.start()
                sends[s][sub][1].wait_recv()
                ccw_ref[rslot, sub] = (
                    ccw_ref[rslot, sub] + pn_ccw[:, lo:lo + nsub]
                )
                d_ccw = make_desc(ccw_ref, rslot, sub, ccw_send_sems,
                                  ccw_recv_sems, left)
                d_ccw.start()
                step_descs.append((d_cw, d_ccw))
            sends[s + 1] = step_descs
            pn_cw = part_cw(c_cw(s + 2))
            pn_ccw = part_ccw(c_ccw(s + 2))

        fslot = (N_DEV - 1) % N_SLOTS
        local_amax = jnp.float32(0)
        for sub in range(N_SUB):
            lo = sub * nsub
            d_cw, d_ccw = sends[N_DEV - 2][sub]
            d_cw.wait_recv()
            y_cw = cw_ref[fslot, sub] + pn_cw[:, lo:lo + nsub]
            out_ref[:, lo:lo + nsub] = y_cw
            d_ccw.wait_recv()
            y_ccw = ccw_ref[fslot, sub] + pn_ccw[:, lo:lo + nsub]
            out_ref[:, n2 + lo:n2 + lo + nsub] = y_ccw
            local_amax = jnp.maximum(
                local_amax,
                jnp.maximum(jnp.max(jnp.abs(y_cw)), jnp.max(jnp.abs(y_ccw))),
            )
        for step in sorted(sends):
            for d_cw, d_ccw in sends[step]:
                d_cw.wait_send()
                d_ccw.wait_send()

        amax_ref[pl.ds(my, 1), :] = jnp.full((1, 128), local_amax, jnp.float32)

        a_sends = []
        for d in range(1, N_DEV):
            dest = lax.rem(my + d, N_DEV)
            snd = pltpu.make_async_remote_copy(
                src_ref=amax_ref.at[pl.ds(my, 1)],
                dst_ref=amax_ref.at[pl.ds(my, 1)],
                send_sem=amax_send_sems.at[d],
                recv_sem=amax_recv_sems.at[d],
                device_id=(dest,),
                device_id_type=pl.DeviceIdType.MESH,
            )
            snd.start()
            a_sends.append(snd)
        for d in range(1, N_DEV):
            src_dev = lax.rem(my - d + 2 * N_DEV, N_DEV)
            rcv = pltpu.make_async_remote_copy(
                src_ref=amax_ref.at[pl.ds(my, 1)],
                dst_ref=amax_ref.at[pl.ds(src_dev, 1)],
                send_sem=amax_send_sems.at[d],
                recv_sem=amax_recv_sems.at[d],
                device_id=(src_dev,),
                device_id_type=pl.DeviceIdType.MESH,
            )
            rcv.wait_recv()
        for snd in a_sends:
            snd.wait_send()

        gmax = jnp.max(amax_ref[...])
        scale = gmax / 127.0
        q = jnp.clip(jnp.round(out_ref[...] / scale), -127.0, 127.0)
        out_ref[...] = q * scale

    return pl.pallas_call(
        body,
        out_shape=jax.ShapeDtypeStruct((chunk, n), jnp.float32),
        in_specs=[
            pl.BlockSpec(memory_space=pltpu.VMEM),
            pl.BlockSpec(memory_space=pltpu.VMEM),
        ],
        out_specs=pl.BlockSpec(memory_space=pltpu.VMEM),
        scratch_shapes=[
            pltpu.VMEM((N_SLOTS, N_SUB, chunk, nsub), jnp.float32),
            pltpu.VMEM((N_SLOTS, N_SUB, chunk, nsub), jnp.float32),
            pltpu.VMEM((N_DEV, 128), jnp.float32),
            pltpu.SemaphoreType.DMA((N_SLOTS * N_SUB,)),
            pltpu.SemaphoreType.DMA((N_SLOTS * N_SUB,)),
            pltpu.SemaphoreType.DMA((N_SLOTS * N_SUB,)),
            pltpu.SemaphoreType.DMA((N_SLOTS * N_SUB,)),
            pltpu.SemaphoreType.DMA((N_DEV,)),
            pltpu.SemaphoreType.DMA((N_DEV,)),
        ],
        compiler_params=pltpu.CompilerParams(
            collective_id=0,
            vmem_limit_bytes=60 * 1024 * 1024,
        ),
    )(x, w_mat)


# device time: 707400 ns/iter; 2.1500x vs baseline; 1.0025x over previous
import jax
import jax.numpy as jnp
from jax import lax
from jax.experimental import pallas as pl
from jax.experimental.pallas import tpu as pltpu

N_DEV = 16
N_SLOTS = 3
N_SUB = 2


def kernel(x, w_mat):
    m, k_per = x.shape
    _, n = w_mat.shape
    chunk = m // N_DEV
    n2 = n // 2
    nsub = n2 // N_SUB

    def body(x_ref, w_ref, out_ref, cw_ref, ccw_ref, amax_ref,
             cw_send_sems, cw_recv_sems, ccw_send_sems, ccw_recv_sems,
             amax_send_sems, amax_recv_sems):
        my = lax.axis_index("i")
        left = lax.rem(my - 1 + N_DEV, N_DEV)
        right = lax.rem(my + 1, N_DEV)

        barrier_sem = pltpu.get_barrier_semaphore()
        pl.semaphore_signal(barrier_sem, inc=1, device_id=(left,),
                            device_id_type=pl.DeviceIdType.MESH)
        pl.semaphore_signal(barrier_sem, inc=1, device_id=(right,),
                            device_id_type=pl.DeviceIdType.MESH)
        pl.semaphore_wait(barrier_sem, 2)

        def part_cw(c):
            return jnp.dot(
                x_ref[pl.ds(c * chunk, chunk), :], w_ref[:, :n2],
                preferred_element_type=jnp.float32,
                precision=lax.Precision.HIGHEST,
            )

        def part_ccw(c):
            return jnp.dot(
                x_ref[pl.ds(c * chunk, chunk), :], w_ref[:, n2:],
                preferred_element_type=jnp.float32,
                precision=lax.Precision.HIGHEST,
            )

        def c_cw(s):
            return lax.rem(my - 1 - s + 2 * N_DEV, N_DEV)

        def c_ccw(s):
            return lax.rem(my + 1 + s, N_DEV)

        def make_desc(ref, slot, sub, send_sems, recv_sems, dest):
            return pltpu.make_async_remote_copy(
                src_ref=ref.at[slot, sub],
                dst_ref=ref.at[(slot + 1) % N_SLOTS, sub],
                send_sem=send_sems.at[slot * N_SUB + sub],
                recv_sem=recv_sems.at[((slot + 1) % N_SLOTS) * N_SUB + sub],
                device_id=(dest,),
                device_id_type=pl.DeviceIdType.MESH,
            )

        sends0 = []
        for sub in range(N_SUB):
            lo = sub * nsub
            cw_ref[0, sub] = jnp.dot(
                x_ref[pl.ds(c_cw(0) * chunk, chunk), :],
                w_ref[:, lo:lo + nsub],
                preferred_element_type=jnp.float32,
                precision=lax.Precision.HIGHEST,
            )
            d_cw = make_desc(cw_ref, 0, sub, cw_send_sems, cw_recv_sems, right)
            d_cw.start()
            ccw_ref[0, sub] = jnp.dot(
                x_ref[pl.ds(c_ccw(0) * chunk, chunk), :],
                w_ref[:, n2 + lo:n2 + lo + nsub],
                preferred_element_type=jnp.float32,
                precision=lax.Precision.HIGHEST,
            )
            d_ccw = make_desc(ccw_ref, 0, sub, ccw_send_sems, ccw_recv_sems,
                              left)
            d_ccw.start()
            sends0.append((d_cw, d_ccw))
        sends = {0: sends0}
        pn_cw = part_cw(c_cw(1))
        pn_ccw = part_ccw(c_ccw(1))

        for s in range(N_DEV - 2):
            rslot = (s + 1) % N_SLOTS
            if s - 2 in sends:
                for d_cw, d_ccw in sends.pop(s - 2):
                    d_cw.wait_send()
                    d_ccw.wait_send()
            step_descs = []
            for sub in range(N_SUB):
                lo = sub * nsub
                sends[s][sub][0].wait_recv()
                cw_ref[rslot, sub] = cw_ref[rslot, sub] + pn_cw[:, lo:lo + nsub]
                d_cw = make_desc(cw_ref, rslot, sub, cw_send_sems,
                                 cw_recv_sems, right)
                d_cw.start()
                sends[s][sub][1].wait_recv()
                ccw_ref[rslot, sub] = (
                    ccw_ref[rslot, sub] + pn_ccw[:, lo:lo + nsub]
                )
                d_ccw = make_desc(ccw_ref, rslot, sub, ccw_send_sems,
                                  ccw_recv_sems, left)
                d_ccw.start()
                step_descs.append((d_cw, d_ccw))
            sends[s + 1] = step_descs
            pn_cw = part_cw(c_cw(s + 2))
            pn_ccw = part_ccw(c_ccw(s + 2))

        fslot = (N_DEV - 1) % N_SLOTS
        local_amax = jnp.float32(0)
        for sub in range(N_SUB):
            lo = sub * nsub
            d_cw, d_ccw = sends[N_DEV - 2][sub]
            d_cw.wait_recv()
            y_cw = cw_ref[fslot, sub] + pn_cw[:, lo:lo + nsub]
            out_ref[:, lo:lo + nsub] = y_cw
            d_ccw.wait_recv()
            y_ccw = ccw_ref[fslot, sub] + pn_ccw[:, lo:lo + nsub]
            out_ref[:, n2 + lo:n2 + lo + nsub] = y_ccw
            local_amax = jnp.maximum(
                local_amax,
                jnp.maximum(jnp.max(jnp.abs(y_cw)), jnp.max(jnp.abs(y_ccw))),
            )
        for step in sorted(sends):
            for d_cw, d_ccw in sends[step]:
                d_cw.wait_send()
                d_ccw.wait_send()

        amax_ref[pl.ds(my, 1), :] = jnp.full((1, 128), local_amax, jnp.float32)

        a_sends = []
        for d in range(1, N_DEV):
            dest = lax.rem(my + d, N_DEV)
            snd = pltpu.make_async_remote_copy(
                src_ref=amax_ref.at[pl.ds(my, 1)],
                dst_ref=amax_ref.at[pl.ds(my, 1)],
                send_sem=amax_send_sems.at[d],
                recv_sem=amax_recv_sems.at[d],
                device_id=(dest,),
                device_id_type=pl.DeviceIdType.MESH,
            )
            snd.start()
            a_sends.append(snd)
        for d in range(1, N_DEV):
            src_dev = lax.rem(my - d + 2 * N_DEV, N_DEV)
            rcv = pltpu.make_async_remote_copy(
                src_ref=amax_ref.at[pl.ds(my, 1)],
                dst_ref=amax_ref.at[pl.ds(src_dev, 1)],
                send_sem=amax_send_sems.at[d],
                recv_sem=amax_recv_sems.at[d],
                device_id=(src_dev,),
                device_id_type=pl.DeviceIdType.MESH,
            )
            rcv.wait_recv()
        for snd in a_sends:
            snd.wait_send()

        gmax = jnp.max(amax_ref[...])
        scale = gmax / 127.0
        q = jnp.clip(jnp.round(out_ref[...] / scale), -127.0, 127.0)
        out_ref[...] = q * scale

    return pl.pallas_call(
        body,
        out_shape=jax.ShapeDtypeStruct((chunk, n), jnp.float32),
        in_specs=[
            pl.BlockSpec(memory_space=pltpu.VMEM),
            pl.BlockSpec(memory_space=pltpu.VMEM),
        ],
        out_specs=pl.BlockSpec(memory_space=pltpu.VMEM),
        scratch_shapes=[
            pltpu.VMEM((N_SLOTS, N_SUB, chunk, nsub), jnp.float32),
            pltpu.VMEM((N_SLOTS, N_SUB, chunk, nsub), jnp.float32),
            pltpu.VMEM((N_DEV, 128), jnp.float32),
            pltpu.SemaphoreType.DMA((N_SLOTS * N_SUB,)),
            pltpu.SemaphoreType.DMA((N_SLOTS * N_SUB,)),
            pltpu.SemaphoreType.DMA((N_SLOTS * N_SUB,)),
            pltpu.SemaphoreType.DMA((N_SLOTS * N_SUB,)),
            pltpu.SemaphoreType.DMA((N_DEV,)),
            pltpu.SemaphoreType.DMA((N_DEV,)),
        ],
        compiler_params=pltpu.CompilerParams(
            collective_id=0,
            vmem_limit_bytes=60 * 1024 * 1024,
        ),
    )(x, w_mat)
